# baseline (speedup 1.0000x reference)
import functools

import jax
import jax.numpy as jnp
from jax import lax
from jax.experimental import pallas as pl
from jax.experimental.pallas import tpu as pltpu
from jax.experimental.pallas import tpu_sc as plsc

NC = 2
NS = 16
L = 16
CH = 128
LQ = 8


def _ceil_to(x, m):
    return (x + m - 1) // m * m


def _count_kernel(n_pad, n_rows, stripe):
    mesh = plsc.VectorSubcoreMesh(core_axis_name="c", subcore_axis_name="s")

    @functools.partial(
        pl.kernel,
        mesh=mesh,
        out_type=jax.ShapeDtypeStruct((NC, n_pad), jnp.float32),
        scratch_types=[
            pltpu.VMEM((n_rows, CH), jnp.int32),
            pltpu.VMEM((CH,), jnp.float32),
            pltpu.VMEM((CH,), jnp.int32),
            pltpu.VMEM((stripe,), jnp.float32),
            pltpu.VMEM_SHARED((n_pad,), jnp.float32),
            pltpu.SemaphoreType.DMA,
        ],
    )
    def k(dst_hbm, out_hbm, idx_v, ones_v, spare_v, stripe_v, cnt_sh, ssem):
        c = lax.axis_index("c")
        s = lax.axis_index("s")
        wid = s * NC + c

        for j in range(CH // L):
            ones_v[pl.ds(j * L, L)] = jnp.ones((L,), jnp.float32)

        def zero_body(j, _):
            stripe_v[pl.ds(j * L, L)] = jnp.zeros((L,), jnp.float32)
            return 0

        lax.fori_loop(0, stripe // L, zero_body, 0)
        pltpu.sync_copy(stripe_v, cnt_sh.at[pl.ds(s * stripe, stripe)])
        pltpu.sync_copy(dst_hbm.at[wid], idx_v)
        plsc.subcore_barrier()

        for _ in range(LQ):
            pltpu.async_copy(dst_hbm.at[wid, 0], spare_v, ssem)

        def body(kk, _):
            pltpu.make_async_copy(ones_v, cnt_sh.at[idx_v.at[kk]], ssem).wait()
            pltpu.async_copy(ones_v, cnt_sh.at[idx_v.at[kk]], ssem, add=True)
            return 0

        lax.fori_loop(0, n_rows, body, 0)

        def drain(kk, _):
            pltpu.make_async_copy(ones_v, cnt_sh.at[idx_v.at[kk]], ssem).wait()
            return 0

        lax.fori_loop(0, LQ, drain, 0)
        plsc.subcore_barrier()
        pltpu.sync_copy(cnt_sh.at[pl.ds(s * stripe, stripe)],
                        out_hbm.at[c, pl.ds(s * stripe, stripe)])

    return k


def _agg_kernel(n_pad, d, n_rows, stripe):
    mesh = plsc.VectorSubcoreMesh(core_axis_name="c", subcore_axis_name="s")

    n_crows = n_rows + LQ

    @functools.partial(
        pl.kernel,
        mesh=mesh,
        compiler_params=pltpu.CompilerParams(needs_layout_passes=False),
        out_type=[jax.ShapeDtypeStruct((NC, n_pad, d), jnp.float32),
                  jax.ShapeDtypeStruct((NC, n_pad), jnp.float32),
                  jax.ShapeDtypeStruct((NC, n_pad, d), jnp.float32)],
        scratch_types=[
            pltpu.VMEM((n_rows, CH), jnp.int32),
            pltpu.VMEM((n_crows, CH), jnp.int32),
            pltpu.VMEM((CH, d), jnp.float32),
            pltpu.VMEM((CH,), jnp.float32),
            pltpu.VMEM((CH,), jnp.int32),
            pltpu.VMEM((stripe,), jnp.float32),
            pltpu.VMEM((stripe,), jnp.float32),
            pltpu.VMEM_SHARED((n_pad, d), jnp.float32),
            pltpu.VMEM_SHARED((n_pad,), jnp.float32),
            pltpu.SemaphoreType.DMA,
            pltpu.SemaphoreType.DMA,
            pltpu.SemaphoreType.DMA,
        ],
    )
    def k(h_hbm, src_hbm, dst_hbm, dstc_hbm, out_hbm, cnt_out_hbm, g_hbm,
          idxs_v, idxd_v, rows_v, ones_v, spare_v, cnt_v, norm_v, agg_sh,
          cnt_sh, gsem, ssem, csem):
        c = lax.axis_index("c")
        s = lax.axis_index("s")
        wid = s * NC + c

        def z_body(j, _):
            r = j // (d // L)
            col = (j % (d // L)) * L
            rows_v[r, pl.ds(col, L)] = jnp.zeros((L,), jnp.float32)
            return 0

        lax.fori_loop(0, CH * (d // L), z_body, 0)
        for j in range(stripe // CH):
            pltpu.sync_copy(rows_v,
                            agg_sh.at[pl.ds(s * stripe + j * CH, CH)])

        def zc_body(j, _):
            cnt_v[pl.ds(j * L, L)] = jnp.zeros((L,), jnp.float32)
            return 0

        lax.fori_loop(0, stripe // L, zc_body, 0)
        pltpu.sync_copy(cnt_v, cnt_sh.at[pl.ds(s * stripe, stripe)])
        for j in range(CH // L):
            ones_v[pl.ds(j * L, L)] = jnp.ones((L,), jnp.float32)
        plsc.subcore_barrier()

        for half in range(2):
            shard = s * NC + half
            pltpu.sync_copy(dstc_hbm.at[shard], idxd_v)
            for _ in range(LQ):
                pltpu.async_copy(dstc_hbm.at[shard, 0], spare_v, csem)

            def c_body(kk, _):
                pltpu.make_async_copy(ones_v, cnt_sh.at[idxd_v.at[kk]],
                                      csem).wait()
                pltpu.async_copy(ones_v, cnt_sh.at[idxd_v.at[kk]], csem,
                                 add=True)
                return 0

            lax.fori_loop(0, n_crows, c_body, 0)

            def c_drain(kk, _):
                pltpu.make_async_copy(ones_v, cnt_sh.at[idxd_v.at[kk]],
                                      csem).wait()
                return 0

            lax.fori_loop(0, LQ, c_drain, 0)
        plsc.subcore_barrier()

        pltpu.sync_copy(cnt_sh.at[pl.ds(s * stripe, stripe)], cnt_v)
        pltpu.sync_copy(cnt_sh.at[pl.ds(s * stripe, stripe)],
                        cnt_out_hbm.at[c, pl.ds(s * stripe, stripe)])
        pltpu.sync_copy(src_hbm.at[wid], idxs_v)
        pltpu.sync_copy(dst_hbm.at[wid], idxd_v.at[pl.ds(0, n_rows)])

        def n_body(j, _):
            x = jnp.maximum(cnt_v[pl.ds(j * L, L)], 1.0)
            i = jnp.int32(0x5F3759DF) - (plsc.bitcast(x, jnp.int32) >> 1)
            y = plsc.bitcast(i, jnp.float32)
            for _ in range(3):
                y = y * (1.5 - 0.5 * x * y * y)
            norm_v[pl.ds(j * L, L)] = y
            return 0

        lax.fori_loop(0, stripe // L, n_body, 0)

        def p_piece(piece, _):
            base = s * stripe + piece * CH
            pltpu.sync_copy(h_hbm.at[pl.ds(base, CH)], rows_v)

            def p_body(r, _):
                scale = plsc.load_gather(
                    norm_v,
                    [jnp.broadcast_to(piece * CH + r, (L,)).astype(jnp.int32)])
                for col in range(d // L):
                    rows_v[r, pl.ds(col * L, L)] = (
                        rows_v[r, pl.ds(col * L, L)] * scale)
                return 0

            lax.fori_loop(0, CH, p_body, 0)
            pltpu.sync_copy(rows_v, g_hbm.at[c, pl.ds(base, CH)])
            return 0

        lax.fori_loop(0, stripe // CH, p_piece, 0)
        plsc.subcore_barrier()

        def body(kk, _):
            pltpu.async_copy(g_hbm.at[c].at[idxs_v.at[kk]], rows_v,
                             gsem).wait()
            pltpu.async_copy(rows_v, agg_sh.at[idxd_v.at[kk]], ssem,
                             add=True).wait()
            return 0

        lax.fori_loop(0, n_rows, body, 0)
        plsc.subcore_barrier()
        pltpu.sync_copy(agg_sh.at[pl.ds(s * stripe, stripe)],
                        out_hbm.at[c, pl.ds(s * stripe, stripe)])

    return k


def _prescale_kernel(n, d, blk):
    def body(cnt_ref, h_ref, g_ref):
        cc = cnt_ref[...]
        norm = lax.rsqrt(jnp.maximum(cc[:, 0:1] + cc[:, 1:2], 1.0))
        g_ref[...] = h_ref[...] * norm

    return pl.pallas_call(
        body,
        grid=(n // blk,),
        in_specs=[
            pl.BlockSpec((blk, 2), lambda i: (i, 0)),
            pl.BlockSpec((blk, d), lambda i: (i, 0)),
        ],
        out_specs=pl.BlockSpec((blk, d), lambda i: (i, 0)),
        out_shape=jax.ShapeDtypeStruct((n, d), jnp.float32),
    )


def _final_kernel(n, d_in, d_out, blk):
    def body(p_ref, cnt_ref, w_ref, b_ref, o_ref):
        agg = p_ref[0] + p_ref[1]
        norm = lax.rsqrt(jnp.maximum(cnt_ref[...], 1.0))
        x = agg * norm
        y = lax.dot_general(x, w_ref[...], (((1,), (1,)), ((), ())),
                            preferred_element_type=jnp.float32)
        o_ref[...] = y + b_ref[...]

    return pl.pallas_call(
        body,
        grid=(n // blk,),
        in_specs=[
            pl.BlockSpec((NC, blk, d_in), lambda i: (0, i, 0)),
            pl.BlockSpec((blk, 1), lambda i: (i, 0)),
            pl.BlockSpec((d_out, d_in), lambda i: (0, 0)),
            pl.BlockSpec((1, d_out), lambda i: (0, 0)),
        ],
        out_specs=pl.BlockSpec((blk, d_out), lambda i: (i, 0)),
        out_shape=jax.ShapeDtypeStruct((n, d_out), jnp.float32),
    )


def kernel(h, edge_index, W, b):
    n, d_in = h.shape
    d_out = W.shape[0]
    e = edge_index.shape[1]
    nw = NC * NS

    src = edge_index[0].astype(jnp.int32)
    dst = edge_index[1].astype(jnp.int32)

    stripe = _ceil_to(n + CH, NS * CH) // NS
    n_pad = stripe * NS
    e_per_tile = _ceil_to((e + nw - 1) // nw, 2 * CH)
    e_pad = e_per_tile * nw
    pad = e_pad - e
    if pad:
        fill = jnp.arange(pad, dtype=jnp.int32)
        src = jnp.concatenate([src, fill % n])
        dst = jnp.concatenate([dst, n + (fill % CH)])
    n_chunks = e_per_tile // CH
    src3 = src.reshape(nw, n_chunks, CH)
    dst3 = dst.reshape(nw, n_chunks, CH)

    spread = jnp.arange(CH, dtype=jnp.int32)

    dst3c = jnp.concatenate(
        [dst3, jnp.broadcast_to(n + spread, (nw, LQ, CH))], axis=1)

    h_pad = jnp.concatenate(
        [h, jnp.zeros((n_pad - n, d_in), jnp.float32)], axis=0)
    partials, counts, _ = _agg_kernel(n_pad, d_in, n_chunks, stripe)(
        h_pad, src3, dst3, dst3c)

    out = _final_kernel(n, d_in, d_out, 1000)(
        partials, counts[0].reshape(n_pad, 1)[:n], W, b.reshape(1, d_out))
    return out


# speedup vs baseline: 1.0460x; 1.0460x over previous
import functools

import jax
import jax.numpy as jnp
from jax import lax
from jax.experimental import pallas as pl
from jax.experimental.pallas import tpu as pltpu
from jax.experimental.pallas import tpu_sc as plsc

NC = 2
NS = 16
L = 16
CH = 128
LQ = 8


def _ceil_to(x, m):
    return (x + m - 1) // m * m


def _count_kernel(n_pad, n_rows, stripe):
    mesh = plsc.VectorSubcoreMesh(core_axis_name="c", subcore_axis_name="s")

    @functools.partial(
        pl.kernel,
        mesh=mesh,
        out_type=jax.ShapeDtypeStruct((NC, n_pad), jnp.float32),
        scratch_types=[
            pltpu.VMEM((n_rows, CH), jnp.int32),
            pltpu.VMEM((CH,), jnp.float32),
            pltpu.VMEM((CH,), jnp.int32),
            pltpu.VMEM((stripe,), jnp.float32),
            pltpu.VMEM_SHARED((n_pad,), jnp.float32),
            pltpu.SemaphoreType.DMA,
        ],
    )
    def k(dst_hbm, out_hbm, idx_v, ones_v, spare_v, stripe_v, cnt_sh, ssem):
        c = lax.axis_index("c")
        s = lax.axis_index("s")
        wid = s * NC + c

        for j in range(CH // L):
            ones_v[pl.ds(j * L, L)] = jnp.ones((L,), jnp.float32)

        def zero_body(j, _):
            stripe_v[pl.ds(j * L, L)] = jnp.zeros((L,), jnp.float32)
            return 0

        lax.fori_loop(0, stripe // L, zero_body, 0)
        pltpu.sync_copy(stripe_v, cnt_sh.at[pl.ds(s * stripe, stripe)])
        pltpu.sync_copy(dst_hbm.at[wid], idx_v)
        plsc.subcore_barrier()

        for _ in range(LQ):
            pltpu.async_copy(dst_hbm.at[wid, 0], spare_v, ssem)

        def body(kk, _):
            pltpu.make_async_copy(ones_v, cnt_sh.at[idx_v.at[kk]], ssem).wait()
            pltpu.async_copy(ones_v, cnt_sh.at[idx_v.at[kk]], ssem, add=True)
            return 0

        lax.fori_loop(0, n_rows, body, 0)

        def drain(kk, _):
            pltpu.make_async_copy(ones_v, cnt_sh.at[idx_v.at[kk]], ssem).wait()
            return 0

        lax.fori_loop(0, LQ, drain, 0)
        plsc.subcore_barrier()
        pltpu.sync_copy(cnt_sh.at[pl.ds(s * stripe, stripe)],
                        out_hbm.at[c, pl.ds(s * stripe, stripe)])

    return k


def _agg_kernel(n_pad, d, n_rows, stripe):
    mesh = plsc.VectorSubcoreMesh(core_axis_name="c", subcore_axis_name="s")

    @functools.partial(
        pl.kernel,
        mesh=mesh,
        out_type=jax.ShapeDtypeStruct((NC, n_pad, d), jnp.float32),
        scratch_types=[
            pltpu.VMEM((n_rows, CH), jnp.int32),
            pltpu.VMEM((n_rows, CH), jnp.int32),
            pltpu.VMEM((CH, d), jnp.float32),
            pltpu.VMEM_SHARED((n_pad, d), jnp.float32),
            pltpu.SemaphoreType.DMA,
            pltpu.SemaphoreType.DMA,
        ],
    )
    def k(g_hbm, src_hbm, dst_hbm, out_hbm, idxs_v, idxd_v, rows_v,
          agg_sh, gsem, ssem):
        c = lax.axis_index("c")
        s = lax.axis_index("s")
        wid = s * NC + c

        def z_body(j, _):
            r = j // (d // L)
            col = (j % (d // L)) * L
            rows_v[r, pl.ds(col, L)] = jnp.zeros((L,), jnp.float32)
            return 0

        lax.fori_loop(0, CH * (d // L), z_body, 0)
        for j in range(stripe // CH):
            pltpu.sync_copy(rows_v,
                            agg_sh.at[pl.ds(s * stripe + j * CH, CH)])
        pltpu.sync_copy(src_hbm.at[wid], idxs_v)
        pltpu.sync_copy(dst_hbm.at[wid], idxd_v)
        plsc.subcore_barrier()

        def body(kk, _):
            pltpu.async_copy(g_hbm.at[idxs_v.at[kk]], rows_v, gsem).wait()
            pltpu.async_copy(rows_v, agg_sh.at[idxd_v.at[kk]], ssem,
                             add=True).wait()
            return 0

        lax.fori_loop(0, n_rows, body, 0)
        plsc.subcore_barrier()
        pltpu.sync_copy(agg_sh.at[pl.ds(s * stripe, stripe)],
                        out_hbm.at[c, pl.ds(s * stripe, stripe)])

    return k


def _prescale_kernel(n, d, blk):
    def body(cnt_ref, h_ref, g_ref):
        cc = cnt_ref[...]
        norm = lax.rsqrt(jnp.maximum(cc[:, 0:1] + cc[:, 1:2], 1.0))
        g_ref[...] = h_ref[...] * norm

    return pl.pallas_call(
        body,
        grid=(n // blk,),
        in_specs=[
            pl.BlockSpec((blk, 2), lambda i: (i, 0)),
            pl.BlockSpec((blk, d), lambda i: (i, 0)),
        ],
        out_specs=pl.BlockSpec((blk, d), lambda i: (i, 0)),
        out_shape=jax.ShapeDtypeStruct((n, d), jnp.float32),
    )


def _final_kernel(n, d_in, d_out, blk):
    def body(p_ref, cnt_ref, w_ref, b_ref, o_ref):
        agg = p_ref[0] + p_ref[1]
        cc = cnt_ref[...]
        norm = lax.rsqrt(jnp.maximum(cc[:, 0:1] + cc[:, 1:2], 1.0))
        x = agg * norm
        y = lax.dot_general(x, w_ref[...], (((1,), (1,)), ((), ())),
                            preferred_element_type=jnp.float32)
        o_ref[...] = y + b_ref[...]

    return pl.pallas_call(
        body,
        grid=(n // blk,),
        in_specs=[
            pl.BlockSpec((NC, blk, d_in), lambda i: (0, i, 0)),
            pl.BlockSpec((blk, 2), lambda i: (i, 0)),
            pl.BlockSpec((d_out, d_in), lambda i: (0, 0)),
            pl.BlockSpec((1, d_out), lambda i: (0, 0)),
        ],
        out_specs=pl.BlockSpec((blk, d_out), lambda i: (i, 0)),
        out_shape=jax.ShapeDtypeStruct((n, d_out), jnp.float32),
    )


def kernel(h, edge_index, W, b):
    n, d_in = h.shape
    d_out = W.shape[0]
    e = edge_index.shape[1]
    nw = NC * NS

    src = edge_index[0].astype(jnp.int32)
    dst = edge_index[1].astype(jnp.int32)

    stripe = _ceil_to(n + CH, NS * CH) // NS
    n_pad = stripe * NS
    e_per_tile = _ceil_to((e + nw - 1) // nw, 2 * CH)
    e_pad = e_per_tile * nw
    pad = e_pad - e
    if pad:
        fill = jnp.arange(pad, dtype=jnp.int32)
        src = jnp.concatenate([src, fill % n])
        dst = jnp.concatenate([dst, n + (fill % CH)])
    n_chunks = e_per_tile // CH
    src3 = src.reshape(nw, n_chunks, CH)
    dst3 = dst.reshape(nw, n_chunks, CH)

    spread = jnp.arange(CH, dtype=jnp.int32)

    dst3c = jnp.concatenate(
        [dst3, jnp.broadcast_to(n + spread, (nw, LQ, CH))], axis=1)
    counts = _count_kernel(n_pad, n_chunks + LQ, stripe)(dst3c)
    cnt_t = counts.T

    g = _prescale_kernel(n, d_in, 1000)(cnt_t[:n], h)

    partials = _agg_kernel(n_pad, d_in, n_chunks, stripe)(g, src3, dst3)

    out = _final_kernel(n, d_in, d_out, 1000)(
        partials, cnt_t[:n], W, b.reshape(1, d_out))
    return out
